# Initial kernel scaffold; baseline (speedup 1.0000x reference)
import jax
import jax.numpy as jnp
from jax.experimental import pallas as pl


def kernel(x, edge_index, edge_attr):
    raise NotImplementedError("write your pallas kernel here")


# speedup vs baseline: 35.5370x; 35.5370x over previous
import functools

import jax
import jax.numpy as jnp
from jax import lax
from jax.experimental import pallas as pl
from jax.experimental.pallas import tpu as pltpu
from jax.experimental.pallas import tpu_sc as plsc

N_NODES = 10000
N_EDGES = 320000
DELTA_X = 0.1

NC = 2
NS = 16
NW = NC * NS
EPW = N_EDGES // NW
N_PAD = 10240
NPT = N_PAD // NS
LANES = 16


def _sc_body(xcol_hbm, src_hbm, dst_hbm, part_hbm,
             xcol_v, sidx_v, didx_v, acc_v, tmp_v, racc_v, shared):
    cid = lax.axis_index("c")
    sid = lax.axis_index("s")
    wid = sid * NC + cid

    pltpu.sync_copy(xcol_hbm, xcol_v)
    base = wid * EPW
    pltpu.sync_copy(src_hbm.at[pl.ds(base, EPW)], sidx_v)
    pltpu.sync_copy(dst_hbm.at[pl.ds(base, EPW)], didx_v)

    zeros = jnp.zeros((LANES,), jnp.float32)

    def zero_body(i, _):
        acc_v[pl.ds(i * LANES, LANES)] = zeros
        return 0
    lax.fori_loop(0, N_PAD // LANES, zero_body, 0)

    def edge_body(i, _):
        off = i * LANES
        s = sidx_v[pl.ds(off, LANES)]
        d = didx_v[pl.ds(off, LANES)]
        vals = plsc.load_gather(xcol_v, [s])
        plsc.addupdate_scatter(acc_v, [d], vals)
        return 0
    lax.fori_loop(0, EPW // LANES, edge_body, 0)

    pltpu.sync_copy(acc_v, shared.at[sid])
    plsc.subcore_barrier()

    nbase = sid * NPT

    def zero2_body(j, _):
        racc_v[pl.ds(j * LANES, LANES)] = zeros
        return 0
    lax.fori_loop(0, NPT // LANES, zero2_body, 0)

    def red_body(t, _):
        pltpu.sync_copy(shared.at[t, pl.ds(nbase, NPT)], tmp_v)

        def add_body(j, _):
            jo = j * LANES
            racc_v[pl.ds(jo, LANES)] = racc_v[pl.ds(jo, LANES)] + tmp_v[pl.ds(jo, LANES)]
            return 0
        lax.fori_loop(0, NPT // LANES, add_body, 0)
        return 0
    lax.fori_loop(0, NS, red_body, 0)

    pltpu.sync_copy(racc_v, part_hbm.at[cid, pl.ds(nbase, NPT)])


@jax.jit
def _sc_scatter(xcol_pad, src, dst):
    mesh = plsc.VectorSubcoreMesh(core_axis_name="c", subcore_axis_name="s")
    return pl.kernel(
        _sc_body,
        out_type=jax.ShapeDtypeStruct((NC, N_PAD), jnp.float32),
        mesh=mesh,
        compiler_params=pltpu.CompilerParams(needs_layout_passes=False),
        scratch_types=[
            pltpu.VMEM((N_PAD,), jnp.float32),
            pltpu.VMEM((EPW,), jnp.int32),
            pltpu.VMEM((EPW,), jnp.int32),
            pltpu.VMEM((N_PAD,), jnp.float32),
            pltpu.VMEM((NPT,), jnp.float32),
            pltpu.VMEM((NPT,), jnp.float32),
            pltpu.VMEM_SHARED((NS, N_PAD), jnp.float32),
        ],
    )(xcol_pad, src, dst)


def _combine_body(p_ref, x_ref, o_ref):
    scale = 1.0 / (DELTA_X * DELTA_X)
    o_ref[...] = (p_ref[0] + p_ref[1] - 2.0 * x_ref[...]) * scale


@jax.jit
def _combine(part, xcol_pad):
    p = part.reshape(NC, N_PAD // 128, 128)
    xr = xcol_pad.reshape(N_PAD // 128, 128)
    out = pl.pallas_call(
        _combine_body,
        out_shape=jax.ShapeDtypeStruct((N_PAD // 128, 128), jnp.float32),
    )(p, xr)
    return out.reshape(N_PAD)


def kernel(x, edge_index, edge_attr):
    xcol = x[:, 0]
    xcol_pad = jnp.pad(xcol, (0, N_PAD - N_NODES))
    src = edge_index[0]
    dst = edge_index[1]
    part = _sc_scatter(xcol_pad, src, dst)
    out = _combine(part, xcol_pad)
    return out[:N_NODES]


# speedup vs baseline: 55.7201x; 1.5679x over previous
import functools

import jax
import jax.numpy as jnp
from jax import lax
from jax.experimental import pallas as pl
from jax.experimental.pallas import tpu as pltpu
from jax.experimental.pallas import tpu_sc as plsc

N_NODES = 10000
N_EDGES = 320000
DELTA_X = 0.1
SCALE = 1.0 / (DELTA_X * DELTA_X)

NC = 2
NS = 16
NW = NC * NS
EPW = N_EDGES // NW
N_PAD = 10240
NPT = N_PAD // NS
LANES = 16


def _sc_body(xcol_hbm, ei_hbm, out0_hbm, out1_hbm,
             xcol_v, sidx_v, didx_v, acc_v, tmp_v, racc_v, shared,
             sem):
    cid = lax.axis_index("c")
    sid = lax.axis_index("s")
    wid = sid * NC + cid
    base = wid * EPW
    r0 = sid * NPT

    cp1 = pltpu.async_copy(ei_hbm.at[pl.ds(base, EPW)], sidx_v, sem)
    cp2 = pltpu.async_copy(ei_hbm.at[pl.ds(N_EDGES + base, EPW)], didx_v, sem)
    cp3 = pltpu.async_copy(xcol_hbm, xcol_v, sem)

    zeros = jnp.zeros((LANES,), jnp.float32)

    def zero_body(i):
        acc_v[pl.ds(i * LANES, LANES)] = zeros
    plsc.parallel_loop(0, N_PAD // LANES, unroll=8)(zero_body)

    cp1.wait()
    cp2.wait()
    cp3.wait()

    def edge_body(i):
        off = i * LANES
        s = sidx_v[pl.ds(off, LANES)]
        d = didx_v[pl.ds(off, LANES)]
        vals = plsc.load_gather(xcol_v, [s])
        plsc.addupdate_scatter(acc_v, [d], vals)
    plsc.parallel_loop(0, EPW // LANES, unroll=8)(edge_body)

    pltpu.sync_copy(acc_v, shared.at[sid])
    plsc.subcore_barrier()

    pltpu.sync_copy(shared.at[0, pl.ds(r0, NPT)], racc_v)

    def red_body(t, _):
        pltpu.sync_copy(shared.at[t, pl.ds(r0, NPT)], tmp_v)

        def add_body(j):
            jo = j * LANES
            racc_v[pl.ds(jo, LANES)] = racc_v[pl.ds(jo, LANES)] + tmp_v[pl.ds(jo, LANES)]
        plsc.parallel_loop(0, NPT // LANES, unroll=8)(add_body)
        return 0
    lax.fori_loop(1, NS, red_body, 0)

    w = (-2.0 * SCALE) * (1.0 - lax.convert_element_type(cid, jnp.float32))

    def ep_body(j):
        jo = j * LANES
        racc_v[pl.ds(jo, LANES)] = (racc_v[pl.ds(jo, LANES)] * SCALE
                                    + w * xcol_v[pl.ds(r0 + jo, LANES)])
    plsc.parallel_loop(0, NPT // LANES, unroll=8)(ep_body)

    @pl.when(cid == 0)
    def _():
        pltpu.sync_copy(racc_v, out0_hbm.at[pl.ds(r0, NPT)])

    @pl.when(cid == 1)
    def _():
        pltpu.sync_copy(racc_v, out1_hbm.at[pl.ds(r0, NPT)])


@jax.jit
def _sc_scatter(xcol_pad, ei):
    mesh = plsc.VectorSubcoreMesh(core_axis_name="c", subcore_axis_name="s")
    return pl.kernel(
        _sc_body,
        out_type=(jax.ShapeDtypeStruct((N_PAD,), jnp.float32),
                  jax.ShapeDtypeStruct((N_PAD,), jnp.float32)),
        mesh=mesh,
        compiler_params=pltpu.CompilerParams(needs_layout_passes=False),
        scratch_types=[
            pltpu.VMEM((N_PAD,), jnp.float32),
            pltpu.VMEM((EPW,), jnp.int32),
            pltpu.VMEM((EPW,), jnp.int32),
            pltpu.VMEM((N_PAD,), jnp.float32),
            pltpu.VMEM((NPT,), jnp.float32),
            pltpu.VMEM((NPT,), jnp.float32),
            pltpu.VMEM_SHARED((NS, N_PAD), jnp.float32),
            pltpu.SemaphoreType.DMA,
        ],
    )(xcol_pad, ei)


def _combine_body(p0_ref, p1_ref, o_ref):
    o_ref[...] = p0_ref[pl.ds(0, N_NODES)] + p1_ref[pl.ds(0, N_NODES)]


@jax.jit
def _combine(p0, p1):
    return pl.pallas_call(
        _combine_body,
        out_shape=jax.ShapeDtypeStruct((N_NODES,), jnp.float32),
    )(p0, p1)


def kernel(x, edge_index, edge_attr):
    xcol = x[:, 0]
    xcol_pad = jnp.pad(xcol, (0, N_PAD - N_NODES))
    p0, p1 = _sc_scatter(xcol_pad, edge_index.reshape(-1))
    return _combine(p0, p1)


# speedup vs baseline: 64.1493x; 1.1513x over previous
import functools

import jax
import jax.numpy as jnp
from jax import lax
from jax.experimental import pallas as pl
from jax.experimental.pallas import tpu as pltpu
from jax.experimental.pallas import tpu_sc as plsc

N_NODES = 10000
N_EDGES = 320000
D_FEAT = 128
DELTA_X = 0.1
SCALE = 1.0 / (DELTA_X * DELTA_X)

NC = 2
NS = 16
NW = NC * NS
N_PAD = 10240
NPT = N_PAD // NS
LANES = 16
ECHUNKS = N_EDGES // 128
CPW = ECHUNKS // NW
CREM = ECHUNKS - CPW * NW
UNROLL = 4


def _sc_body(x_hbm, ei_hbm, out0_hbm, out1_hbm,
             xcol_sh, xcol_v, gidx_v, gath_v, eiv, acc_v, tmp_v, racc_v,
             shared, sem):
    cid = lax.axis_index("c")
    sid = lax.axis_index("s")
    wid = sid * NC + cid
    r0 = sid * NPT
    t0 = wid * CPW

    cp1 = pltpu.async_copy(ei_hbm.at[pl.ds(t0, CPW)], eiv.at[pl.ds(0, CPW)], sem)

    @pl.when(wid < CREM)
    def _():
        pltpu.async_copy(ei_hbm.at[pl.ds(NW * CPW + wid, 1)],
                         eiv.at[pl.ds(CPW, 1)], sem).wait()

    lanes = lax.iota(jnp.int32, LANES)
    for k in range(NPT // 128):
        for c in range(128 // LANES):
            j0 = k * 128 + c * LANES
            n = jnp.minimum(lanes + (r0 + j0), N_NODES - 1)
            gidx_v[k, pl.ds(c * LANES, LANES)] = n * D_FEAT
    gcps = [pltpu.async_copy(x_hbm.at[gidx_v.at[k]],
                             gath_v.at[pl.ds(k * 128, 128)], sem)
            for k in range(NPT // 128)]

    zeros = jnp.zeros((LANES,), jnp.float32)

    def zero_body(i):
        acc_v[pl.ds(i * LANES, LANES)] = zeros
    plsc.parallel_loop(0, N_PAD // LANES, unroll=8)(zero_body)

    cp1.wait()
    for cp in gcps:
        cp.wait()

    pltpu.sync_copy(gath_v, xcol_sh.at[pl.ds(r0, NPT)])
    plsc.subcore_barrier()
    pltpu.sync_copy(xcol_sh, xcol_v)

    def edge_chunk(t):
        for c in range(128 // LANES):
            s = eiv[t, 0, pl.ds(c * LANES, LANES)]
            d = eiv[t, 1, pl.ds(c * LANES, LANES)]
            vals = plsc.load_gather(xcol_v, [s])
            plsc.addupdate_scatter(acc_v, [d], vals)
    plsc.parallel_loop(0, CPW, unroll=UNROLL)(edge_chunk)

    @pl.when(wid < CREM)
    def _():
        edge_chunk(CPW)

    pltpu.sync_copy(acc_v, shared.at[sid])
    plsc.subcore_barrier()

    pltpu.sync_copy(shared.at[0, pl.ds(r0, NPT)], racc_v)

    def red_body(t, _):
        pltpu.sync_copy(shared.at[t, pl.ds(r0, NPT)], tmp_v)

        def add_body(j):
            jo = j * LANES
            racc_v[pl.ds(jo, LANES)] = racc_v[pl.ds(jo, LANES)] + tmp_v[pl.ds(jo, LANES)]
        plsc.parallel_loop(0, NPT // LANES, unroll=8)(add_body)
        return 0
    lax.fori_loop(1, NS, red_body, 0)

    w = (-2.0 * SCALE) * (1.0 - lax.convert_element_type(cid, jnp.float32))

    def ep_body(j):
        jo = j * LANES
        racc_v[pl.ds(jo, LANES)] = (racc_v[pl.ds(jo, LANES)] * SCALE
                                    + w * gath_v[pl.ds(jo, LANES)])
    plsc.parallel_loop(0, NPT // LANES, unroll=8)(ep_body)

    @pl.when(cid == 0)
    def _():
        pltpu.sync_copy(racc_v, out0_hbm.at[pl.ds(r0, NPT)])

    @pl.when(cid == 1)
    def _():
        pltpu.sync_copy(racc_v, out1_hbm.at[pl.ds(r0, NPT)])


@jax.jit
def _sc_scatter(x_flat, ei_blk):
    mesh = plsc.VectorSubcoreMesh(core_axis_name="c", subcore_axis_name="s")
    return pl.kernel(
        _sc_body,
        out_type=(jax.ShapeDtypeStruct((N_PAD,), jnp.float32),
                  jax.ShapeDtypeStruct((N_PAD,), jnp.float32)),
        mesh=mesh,
        compiler_params=pltpu.CompilerParams(needs_layout_passes=False),
        scratch_types=[
            pltpu.VMEM_SHARED((N_PAD,), jnp.float32),
            pltpu.VMEM((N_PAD,), jnp.float32),
            pltpu.VMEM((NPT // 128, 128), jnp.int32),
            pltpu.VMEM((NPT,), jnp.float32),
            pltpu.VMEM((CPW + 1, 2, 128), jnp.int32),
            pltpu.VMEM((N_PAD,), jnp.float32),
            pltpu.VMEM((NPT,), jnp.float32),
            pltpu.VMEM((NPT,), jnp.float32),
            pltpu.VMEM_SHARED((NS, N_PAD), jnp.float32),
            pltpu.SemaphoreType.DMA,
        ],
    )(x_flat, ei_blk)


def _combine_body(p0_ref, p1_ref, o_ref):
    o_ref[...] = p0_ref[pl.ds(0, N_NODES)] + p1_ref[pl.ds(0, N_NODES)]


@jax.jit
def _combine(p0, p1):
    return pl.pallas_call(
        _combine_body,
        out_shape=jax.ShapeDtypeStruct((N_NODES,), jnp.float32),
    )(p0, p1)


def kernel(x, edge_index, edge_attr):
    x_flat = x.reshape(-1)
    ei_blk = jnp.transpose(edge_index.reshape(2, ECHUNKS, 128), (1, 0, 2))
    p0, p1 = _sc_scatter(x_flat, ei_blk)
    return _combine(p0, p1)
